# baseline (device time: 59358 ns/iter reference)
import jax
import jax.numpy as jnp
from jax import lax
from jax.experimental import pallas as pl
from jax.experimental.pallas import tpu as pltpu

N_DEV = 4
G = 4


def kernel(x, Wg, Wu, Wd):
    m, _ = x.shape
    d = Wd.shape[1]
    c = m // N_DEV
    hc = c // G

    def body(x_ref, wg_ref, wu_ref, wd_ref, out_ref,
             wgb_ref, wub_ref, wdb_ref, rs_send, rs_buf,
             rs_send_sems, rs_recv_sems, ag_send_sems, ag_recv_sems):
        my = lax.axis_index("i")

        barrier_sem = pltpu.get_barrier_semaphore()
        for off in (1, 2, 3):
            pl.semaphore_signal(
                barrier_sem, inc=1,
                device_id=((my + off) % N_DEV,),
                device_id_type=pl.DeviceIdType.MESH,
            )
        pl.semaphore_wait(barrier_sem, 3)

        wgb_ref[...] = wg_ref[...].astype(jnp.bfloat16)
        wub_ref[...] = wu_ref[...].astype(jnp.bfloat16)
        wdb_ref[...] = wd_ref[...].astype(jnp.bfloat16)

        def partial_rows(r0):
            xj = x_ref[pl.ds(r0, hc), :].astype(jnp.bfloat16)
            gj = jnp.dot(xj, wgb_ref[...], preferred_element_type=jnp.float32)
            uj = jnp.dot(xj, wub_ref[...], preferred_element_type=jnp.float32)
            hj = (gj * (uj * jax.nn.sigmoid(uj))).astype(jnp.bfloat16)
            return jnp.dot(hj, wdb_ref[...], preferred_element_type=jnp.float32)

        rs_rdmas = {}

        def remote(g):
            for off in (1, 2, 3):
                j = (my + off) % N_DEV
                slot = (off - 1) * G + g
                rs_send[slot] = partial_rows(j * c + g * hc).astype(jnp.bfloat16)
                rdma = pltpu.make_async_remote_copy(
                    src_ref=rs_send.at[slot],
                    dst_ref=rs_buf.at[slot],
                    send_sem=rs_send_sems.at[slot],
                    recv_sem=rs_recv_sems.at[slot],
                    device_id=(j,),
                    device_id_type=pl.DeviceIdType.MESH,
                )
                rdma.start()
                rs_rdmas[slot] = rdma

        ag_rdmas = []

        def reduce_and_broadcast(g, acc):
            for off in (1, 2, 3):
                slot = (off - 1) * G + g
                rs_rdmas[slot].wait_recv()
                acc = acc + rs_buf[slot].astype(jnp.float32)
            r0 = my * c + g * hc
            out_ref[pl.ds(r0, hc), :] = acc.astype(jnp.bfloat16)
            for off in (1, 2, 3):
                j = (my + off) % N_DEV
                slot = (off - 1) * G + g
                rdma = pltpu.make_async_remote_copy(
                    src_ref=out_ref.at[pl.ds(r0, hc), :],
                    dst_ref=out_ref.at[pl.ds(r0, hc), :],
                    send_sem=ag_send_sems.at[slot],
                    recv_sem=ag_recv_sems.at[slot],
                    device_id=(j,),
                    device_id_type=pl.DeviceIdType.MESH,
                )
                rdma.start()
                ag_rdmas.append(rdma)

        remote(0)
        acc = partial_rows(my * c)
        for g in range(1, G):
            remote(g)
            reduce_and_broadcast(g - 1, acc)
            acc = partial_rows(my * c + g * hc)
        reduce_and_broadcast(G - 1, acc)

        for r in ag_rdmas:
            r.wait_recv()
        for r in list(rs_rdmas.values()) + ag_rdmas:
            r.wait_send()

    return pl.pallas_call(
        body,
        out_shape=jax.ShapeDtypeStruct((m, d), jnp.bfloat16),
        in_specs=[pl.BlockSpec(memory_space=pltpu.VMEM)] * 4,
        out_specs=pl.BlockSpec(memory_space=pltpu.VMEM),
        scratch_shapes=[
            pltpu.VMEM(Wg.shape, jnp.bfloat16),
            pltpu.VMEM(Wu.shape, jnp.bfloat16),
            pltpu.VMEM(Wd.shape, jnp.bfloat16),
            pltpu.VMEM((3 * G, hc, d), jnp.bfloat16),
            pltpu.VMEM((3 * G, hc, d), jnp.bfloat16),
            pltpu.SemaphoreType.DMA((3 * G,)),
            pltpu.SemaphoreType.DMA((3 * G,)),
            pltpu.SemaphoreType.DMA((3 * G,)),
            pltpu.SemaphoreType.DMA((3 * G,)),
        ],
        compiler_params=pltpu.CompilerParams(
            collective_id=0, vmem_limit_bytes=41 * 1024 * 1024,
        ),
    )(x, Wg, Wu, Wd)


# device time: 45004 ns/iter; 1.3189x vs baseline; 1.3189x over previous
import jax
import jax.numpy as jnp
from jax import lax
from jax.experimental import pallas as pl
from jax.experimental.pallas import tpu as pltpu

N_DEV = 4
G = 2


def kernel(x, Wg, Wu, Wd):
    m, _ = x.shape
    d = Wd.shape[1]
    c = m // N_DEV
    hc = c // G

    def body(x_ref, wg_ref, wu_ref, wd_ref, out_ref,
             wgb_ref, wub_ref, wdb_ref, rs_send, rs_buf,
             rs_send_sems, rs_recv_sems, ag_send_sems, ag_recv_sems):
        my = lax.axis_index("i")

        barrier_sem = pltpu.get_barrier_semaphore()
        for off in (1, 2, 3):
            pl.semaphore_signal(
                barrier_sem, inc=1,
                device_id=((my + off) % N_DEV,),
                device_id_type=pl.DeviceIdType.MESH,
            )
        pl.semaphore_wait(barrier_sem, 3)

        wgb_ref[...] = wg_ref[...].astype(jnp.bfloat16)
        wub_ref[...] = wu_ref[...].astype(jnp.bfloat16)
        wdb_ref[...] = wd_ref[...].astype(jnp.bfloat16)

        def partial_rows(r0):
            xj = x_ref[pl.ds(r0, hc), :].astype(jnp.bfloat16)
            gj = jnp.dot(xj, wgb_ref[...], preferred_element_type=jnp.float32)
            uj = jnp.dot(xj, wub_ref[...], preferred_element_type=jnp.float32)
            hj = (gj * (uj * jax.nn.sigmoid(uj))).astype(jnp.bfloat16)
            return jnp.dot(hj, wdb_ref[...], preferred_element_type=jnp.float32)

        rs_rdmas = {}

        def remote(g):
            for off in (1, 2, 3):
                j = (my + off) % N_DEV
                slot = (off - 1) * G + g
                rs_send[slot] = partial_rows(j * c + g * hc).astype(jnp.bfloat16)
                rdma = pltpu.make_async_remote_copy(
                    src_ref=rs_send.at[slot],
                    dst_ref=rs_buf.at[slot],
                    send_sem=rs_send_sems.at[slot],
                    recv_sem=rs_recv_sems.at[slot],
                    device_id=(j,),
                    device_id_type=pl.DeviceIdType.MESH,
                )
                rdma.start()
                rs_rdmas[slot] = rdma

        ag_rdmas = []

        def reduce_and_broadcast(g, acc):
            for off in (1, 2, 3):
                slot = (off - 1) * G + g
                rs_rdmas[slot].wait_recv()
                acc = acc + rs_buf[slot].astype(jnp.float32)
            r0 = my * c + g * hc
            out_ref[pl.ds(r0, hc), :] = acc.astype(jnp.bfloat16)
            for off in (1, 2, 3):
                j = (my + off) % N_DEV
                slot = (off - 1) * G + g
                rdma = pltpu.make_async_remote_copy(
                    src_ref=out_ref.at[pl.ds(r0, hc), :],
                    dst_ref=out_ref.at[pl.ds(r0, hc), :],
                    send_sem=ag_send_sems.at[slot],
                    recv_sem=ag_recv_sems.at[slot],
                    device_id=(j,),
                    device_id_type=pl.DeviceIdType.MESH,
                )
                rdma.start()
                ag_rdmas.append(rdma)

        remote(0)
        acc = partial_rows(my * c)
        for g in range(1, G):
            remote(g)
            reduce_and_broadcast(g - 1, acc)
            acc = partial_rows(my * c + g * hc)
        reduce_and_broadcast(G - 1, acc)

        for r in ag_rdmas:
            r.wait_recv()
        for r in list(rs_rdmas.values()) + ag_rdmas:
            r.wait_send()

    return pl.pallas_call(
        body,
        out_shape=jax.ShapeDtypeStruct((m, d), jnp.bfloat16),
        in_specs=[pl.BlockSpec(memory_space=pltpu.VMEM)] * 4,
        out_specs=pl.BlockSpec(memory_space=pltpu.VMEM),
        scratch_shapes=[
            pltpu.VMEM(Wg.shape, jnp.bfloat16),
            pltpu.VMEM(Wu.shape, jnp.bfloat16),
            pltpu.VMEM(Wd.shape, jnp.bfloat16),
            pltpu.VMEM((3 * G, hc, d), jnp.bfloat16),
            pltpu.VMEM((3 * G, hc, d), jnp.bfloat16),
            pltpu.SemaphoreType.DMA((3 * G,)),
            pltpu.SemaphoreType.DMA((3 * G,)),
            pltpu.SemaphoreType.DMA((3 * G,)),
            pltpu.SemaphoreType.DMA((3 * G,)),
        ],
        compiler_params=pltpu.CompilerParams(
            collective_id=0, vmem_limit_bytes=41 * 1024 * 1024,
        ),
    )(x, Wg, Wu, Wd)
